# baseline (speedup 1.0000x reference)
import jax
import jax.numpy as jnp
from jax.experimental import pallas as pl
from jax.experimental.pallas import tpu as pltpu

_R = 8


def _argmin_chunk_body(x_ref, o_ref, mv_ref, mi_ref):
    k = pl.program_id(1)
    nk = pl.num_programs(1)
    ch, d = x_ref.shape[1], x_ref.shape[2]
    ng = ch // _R

    @pl.when(k == 0)
    def _init():
        mv_ref[...] = jnp.full((_R, d), jnp.inf, jnp.float32)
        mi_ref[...] = jnp.zeros((_R, d), jnp.int32)

    x3 = x_ref[0].reshape(ng, _R, d)
    mv = mv_ref[...]
    mi = mi_ref[...]
    base = k * ng
    for g in range(ng):
        v = x3[g]
        mask = v < mv
        mv = jnp.where(mask, v, mv)
        mi = jnp.where(mask, (base + g).astype(jnp.int32), mi)
    mv_ref[...] = mv
    mi_ref[...] = mi

    @pl.when(k == nk - 1)
    def _fin():
        m = jnp.min(mv, axis=0)
        sub = jax.lax.broadcasted_iota(jnp.int32, (_R, d), 0)
        full = mi * _R + sub
        big = jnp.int32(2**30)
        cand = jnp.where(mv == m[None], full, big)
        o_ref[0, 0, :] = jnp.min(cand, axis=0)


def kernel(x):
    B, N, D = x.shape
    CH = 1024 if N % 1024 == 0 else N
    out = pl.pallas_call(
        _argmin_chunk_body,
        grid=(B, N // CH),
        in_specs=[pl.BlockSpec((1, CH, D), lambda b, k: (b, k, 0))],
        out_specs=pl.BlockSpec((1, 1, D), lambda b, k: (b, 0, 0)),
        out_shape=jax.ShapeDtypeStruct((B, 1, D), jnp.int32),
        scratch_shapes=[
            pltpu.VMEM((_R, D), jnp.float32),
            pltpu.VMEM((_R, D), jnp.int32),
        ],
        compiler_params=pltpu.CompilerParams(
            dimension_semantics=("arbitrary", "arbitrary"),
        ),
    )(x)
    return out.reshape(B, D).astype(jnp.int64)
